# baseline (speedup 1.0000x reference)
import functools

import jax
import jax.numpy as jnp
from jax.experimental import pallas as pl
from jax.experimental.pallas import tpu as pltpu

CODE_DIM = 512
NB_CODE = 8192
WIDTH = 512
IN_DIM = 159
OUT_DIM = 159
TEMP = 0.1
BATCH = 1024
T_IN = 16
T_ENC = 4

_PREC = jax.lax.Precision.HIGHEST


def _mm(a, b):
    return jax.lax.dot_general(a, b, (((1,), (0,)), ((), ())),
                               preferred_element_type=jnp.float32,
                               precision=_PREC)


def _mm_t(a, b):
    return jax.lax.dot_general(a, b, (((1,), (1,)), ((), ())),
                               preferred_element_type=jnp.float32,
                               precision=_PREC)


def _conv_time(xs, w_ref, b_ref, k, stride=1, pad=0, relu_out=False):
    t_in = len(xs)
    t_out = (t_in + 2 * pad - k) // stride + 1
    outs = []
    bias = b_ref[:]
    for t in range(t_out):
        acc = None
        for j in range(k):
            ti = t * stride - pad + j
            if 0 <= ti < t_in:
                p = _mm(xs[ti], w_ref[j])
                acc = p if acc is None else acc + p
        acc = acc + bias
        if relu_out:
            acc = jnp.maximum(acc, 0.0)
        outs.append(acc)
    return outs


def _encoder_body(x_ref, wi_ref, bi_ref,
                  wd1_ref, bd1_ref, w11_ref, b11_ref, w12_ref, b12_ref,
                  wd2_ref, bd2_ref, w21_ref, b21_ref, w22_ref, b22_ref,
                  wo_ref, bo_ref, out_ref):
    xs = [x_ref[t] for t in range(T_IN)]
    h = _conv_time(xs, wi_ref, bi_ref, k=3, pad=1, relu_out=True)
    for (wd, bd, w1, b1, w2, b2) in (
            (wd1_ref, bd1_ref, w11_ref, b11_ref, w12_ref, b12_ref),
            (wd2_ref, bd2_ref, w21_ref, b21_ref, w22_ref, b22_ref)):
        hd = _conv_time(h, wd, bd, k=4, stride=2, pad=1)
        r = [jnp.maximum(a, 0.0) for a in hd]
        r = _conv_time(r, w1, b1, k=3, pad=1, relu_out=True)
        r = _conv_time(r, w2, b2, k=1)
        h = [a + b for a, b in zip(hd, r)]
    h = _conv_time(h, wo_ref, bo_ref, k=3, pad=1)
    for t in range(T_ENC):
        out_ref[t] = h[t]


def _run_encoder(x_tbc, enc_params, bblk):
    (w_in, b_in), blocks, (w_out, b_out) = enc_params
    wlist = []
    for w, b in ((w_in, b_in),
                 *[lw for blk in blocks for lw in blk],
                 (w_out, b_out)):
        wlist.append(jnp.transpose(w, (2, 1, 0)))
        wlist.append(b.reshape(1, -1))
    grid = (BATCH // bblk,)
    in_specs = [pl.BlockSpec((T_IN, bblk, IN_DIM), lambda i: (0, i, 0))]
    for w in wlist:
        in_specs.append(pl.BlockSpec(w.shape, lambda i, n=w.ndim: (0,) * n))
    out_spec = pl.BlockSpec((T_ENC, bblk, CODE_DIM), lambda i: (0, i, 0))
    return pl.pallas_call(
        _encoder_body,
        grid=grid,
        in_specs=in_specs,
        out_specs=out_spec,
        out_shape=jax.ShapeDtypeStruct((T_ENC, BATCH, CODE_DIM), jnp.float32),
    )(x_tbc, *wlist)


def _quant_body(xf_ref, cb_ref, idx_ref, xq_ref, *, tblk, cblk):
    xf = xf_ref[:]
    x2 = jnp.sum(xf * xf, axis=1, keepdims=True)
    xm2 = xf * (-2.0)
    ones = jnp.ones((tblk, 1), jnp.float32)
    xaug = jnp.concatenate([xm2, ones], axis=1)
    best = jnp.full((tblk, 1), jnp.inf, jnp.float32)
    barg = jnp.zeros((tblk, 1), jnp.int32)
    iota = jax.lax.broadcasted_iota(jnp.int32, (tblk, cblk), 1)
    for c in range(NB_CODE // cblk):
        cb = cb_ref[c * cblk:(c + 1) * cblk, :]
        c2 = jnp.sum(cb * cb, axis=1, keepdims=True)
        caug = jnp.concatenate([cb, c2], axis=1)
        dist = x2 + _mm_t(xaug, caug)
        lmin = jnp.min(dist, axis=1, keepdims=True)
        cand = jnp.where(dist == lmin, iota, cblk)
        larg = jnp.min(cand, axis=1, keepdims=True) + c * cblk
        better = lmin < best
        best = jnp.where(better, lmin, best)
        barg = jnp.where(better, larg, barg)
    idx_ref[:] = barg
    acc = None
    for c in range(NB_CODE // cblk):
        rel = barg - c * cblk
        oh = (iota == rel).astype(jnp.float32)
        p = _mm(oh, cb_ref[c * cblk:(c + 1) * cblk, :])
        acc = p if acc is None else acc + p
    xq_ref[:] = acc


def _run_quantizer(xf, codebook, tblk, cblk):
    ntok = xf.shape[0]
    grid = (ntok // tblk,)
    body = functools.partial(_quant_body, tblk=tblk, cblk=cblk)
    return pl.pallas_call(
        body,
        grid=grid,
        in_specs=[pl.BlockSpec((tblk, CODE_DIM), lambda i: (i, 0)),
                  pl.BlockSpec((NB_CODE, CODE_DIM), lambda i: (0, 0))],
        out_specs=[pl.BlockSpec((tblk, 1), lambda i: (i, 0)),
                   pl.BlockSpec((tblk, CODE_DIM), lambda i: (i, 0))],
        out_shape=[jax.ShapeDtypeStruct((ntok, 1), jnp.int32),
                   jax.ShapeDtypeStruct((ntok, CODE_DIM), jnp.float32)],
    )(xf, codebook)


def _decoder_body(x_ref, wi_ref, bi_ref,
                  w11_ref, b11_ref, w12_ref, b12_ref, wu1_ref, bu1_ref,
                  w21_ref, b21_ref, w22_ref, b22_ref, wu2_ref, bu2_ref,
                  wm_ref, bm_ref, wo_ref, bo_ref, out_ref):
    xs = [x_ref[t] for t in range(T_ENC)]
    h = _conv_time(xs, wi_ref, bi_ref, k=3, pad=1, relu_out=True)
    for (w1, b1, w2, b2, wu, bu) in (
            (w11_ref, b11_ref, w12_ref, b12_ref, wu1_ref, bu1_ref),
            (w21_ref, b21_ref, w22_ref, b22_ref, wu2_ref, bu2_ref)):
        r = [jnp.maximum(a, 0.0) for a in h]
        r = _conv_time(r, w1, b1, k=3, pad=1, relu_out=True)
        r = _conv_time(r, w2, b2, k=1)
        h = [a + b for a, b in zip(h, r)]
        h = [a for a in h for _ in range(2)]
        h = _conv_time(h, wu, bu, k=3, pad=1)
    h = _conv_time(h, wm_ref, bm_ref, k=3, pad=1, relu_out=True)
    h = _conv_time(h, wo_ref, bo_ref, k=3, pad=1)
    for t in range(T_IN):
        out_ref[t] = h[t]


def _run_decoder(xq_tbc, dec_params, bblk):
    (w_in, b_in), blocks, (w_mid, b_mid), (w_out, b_out) = dec_params
    wlist = []
    for w, b in ((w_in, b_in),
                 *[lw for blk in blocks for lw in blk],
                 (w_mid, b_mid), (w_out, b_out)):
        wlist.append(jnp.transpose(w, (2, 1, 0)))
        wlist.append(b.reshape(1, -1))
    grid = (BATCH // bblk,)
    in_specs = [pl.BlockSpec((T_ENC, bblk, CODE_DIM), lambda i: (0, i, 0))]
    for w in wlist:
        in_specs.append(pl.BlockSpec(w.shape, lambda i, n=w.ndim: (0,) * n))
    out_spec = pl.BlockSpec((T_IN, bblk, OUT_DIM), lambda i: (0, i, 0))
    return pl.pallas_call(
        _decoder_body,
        grid=grid,
        in_specs=in_specs,
        out_specs=out_spec,
        out_shape=jax.ShapeDtypeStruct((T_IN, BATCH, OUT_DIM), jnp.float32),
    )(xq_tbc, *wlist)


def _norm_rows(z):
    return z / (jnp.sqrt(jnp.sum(z * z, axis=1, keepdims=True)) + 1e-12)


def _sim_loss(zi_n, zj_n, mask):
    sim = _mm_t(zj_n, zi_n) / TEMP
    e = jnp.exp(sim)
    sp = jnp.sum(e * mask, axis=1, keepdims=True)
    sn = jnp.sum(e * (1.0 - mask), axis=1, keepdims=True)
    lp = jnp.log(sp / sn)
    rc = jnp.sum(mask, axis=1, keepdims=True)
    return -jnp.mean(rc * lp)


def _losses_body(te_ref, tw_ref, tb_ref, au_ref, g_ref, vc_ref, vr_ref,
                 o1_ref, o2_ref, o3_ref):
    b = te_ref.shape[0]
    text = _mm_t(te_ref[:], tw_ref[:]) + tb_ref[:]
    tn = _norm_rows(text)
    an = _norm_rows(au_ref[:])
    g1n = _norm_rows(g_ref[0])
    g2n = _norm_rows(g_ref[1])
    g3n = _norm_rows(g_ref[2])
    g4n = _norm_rows(g_ref[3])
    eye = (jax.lax.broadcasted_iota(jnp.int32, (b, b), 0)
           == jax.lax.broadcasted_iota(jnp.int32, (b, b), 1)).astype(jnp.float32)
    style = (vc_ref[:] == vr_ref[:]).astype(jnp.float32)
    o1_ref[:] = jnp.reshape(_sim_loss(tn, g1n, eye), (1, 1))
    o2_ref[:] = jnp.reshape(_sim_loss(an, g2n, eye), (1, 1))
    o3_ref[:] = jnp.reshape(_sim_loss(g3n, g4n, style), (1, 1))


def _run_losses(text_emb, text_W, text_b, audio, x_enc_tbc, vid):
    vid_i = vid.astype(jnp.int32)
    vc = vid_i.reshape(BATCH, 1)
    vr = vid_i.reshape(1, BATCH)
    scal = jax.ShapeDtypeStruct((1, 1), jnp.float32)
    return pl.pallas_call(
        _losses_body,
        out_shape=[scal, scal, scal],
    )(text_emb, text_W, text_b.reshape(1, -1), audio, x_enc_tbc, vc, vr)


def _stats_body(xf_ref, xq_ref, idx_ref, commit_ref, perp_ref, *, cblk):
    ntok = xf_ref.shape[0]
    d = xf_ref[:] - xq_ref[:]
    commit_ref[:] = jnp.reshape(jnp.sum(d * d) / (ntok * CODE_DIM), (1, 1))
    idx = idx_ref[:]
    ent = jnp.zeros((), jnp.float32)
    iota = jax.lax.broadcasted_iota(jnp.int32, (ntok, cblk), 1)
    inv = 1.0 / ntok
    for c in range(NB_CODE // cblk):
        eq = (idx == (iota + c * cblk)).astype(jnp.float32)
        cnt = jnp.sum(eq, axis=0)
        p = cnt * inv
        ent = ent + jnp.sum(p * jnp.log(p + 1e-10))
    perp_ref[:] = jnp.reshape(jnp.exp(-ent), (1, 1))


def _run_stats(xf, xq, idx, cblk=1024):
    scal = jax.ShapeDtypeStruct((1, 1), jnp.float32)
    body = functools.partial(_stats_body, cblk=cblk)
    return pl.pallas_call(body, out_shape=[scal, scal])(xf, xq, idx)


def kernel(gesture, audio_embeddings, text_embeddings, vid_indices, codebook,
           enc_params, dec_params, text_W, text_b):
    x_tbc = jnp.transpose(gesture.astype(jnp.float32), (1, 0, 2))
    x_enc = _run_encoder(x_tbc, enc_params, bblk=256)

    ntok = T_ENC * BATCH
    xf = x_enc.reshape(ntok, CODE_DIM)
    idx, xq = _run_quantizer(xf, codebook, tblk=512, cblk=1024)

    commit, perp = _run_stats(xf, xq, idx)

    gt_loss, ga_loss, gs_loss = _run_losses(
        text_embeddings, text_W, text_b, audio_embeddings, x_enc, vid_indices)

    xq_tbc = xq.reshape(T_ENC, BATCH, CODE_DIM)
    x_dec = _run_decoder(xq_tbc, dec_params, bblk=256)
    x_out = jnp.transpose(x_dec, (1, 0, 2))

    return (x_out, commit[0, 0], perp[0, 0],
            gt_loss[0, 0], ga_loss[0, 0], gs_loss[0, 0])


# speedup vs baseline: 3.0484x; 3.0484x over previous
import functools

import jax
import jax.numpy as jnp
from jax.experimental import pallas as pl
from jax.experimental.pallas import tpu as pltpu

CODE_DIM = 512
NB_CODE = 8192
WIDTH = 512
IN_DIM = 159
OUT_DIM = 159
TEMP = 0.1
BATCH = 1024
T_IN = 16
T_ENC = 4

_PREC = jax.lax.Precision.DEFAULT


def _mm(a, b, prec=_PREC):
    return jax.lax.dot_general(a, b, (((1,), (0,)), ((), ())),
                               preferred_element_type=jnp.float32,
                               precision=prec)


def _mm_t(a, b, prec=_PREC):
    return jax.lax.dot_general(a, b, (((1,), (1,)), ((), ())),
                               preferred_element_type=jnp.float32,
                               precision=prec)


def _conv_time(xs, w_ref, b_ref, k, stride=1, pad=0, relu_out=False):
    t_in = len(xs)
    t_out = (t_in + 2 * pad - k) // stride + 1
    outs = []
    bias = b_ref[:]
    for t in range(t_out):
        acc = None
        for j in range(k):
            ti = t * stride - pad + j
            if 0 <= ti < t_in:
                p = _mm(xs[ti], w_ref[j])
                acc = p if acc is None else acc + p
        acc = acc + bias
        if relu_out:
            acc = jnp.maximum(acc, 0.0)
        outs.append(acc)
    return outs


def _encoder_body(x_ref, wi_ref, bi_ref,
                  wd1_ref, bd1_ref, w11_ref, b11_ref, w12_ref, b12_ref,
                  wd2_ref, bd2_ref, w21_ref, b21_ref, w22_ref, b22_ref,
                  wo_ref, bo_ref, out_ref):
    xs = [x_ref[t] for t in range(T_IN)]
    h = _conv_time(xs, wi_ref, bi_ref, k=3, pad=1, relu_out=True)
    for (wd, bd, w1, b1, w2, b2) in (
            (wd1_ref, bd1_ref, w11_ref, b11_ref, w12_ref, b12_ref),
            (wd2_ref, bd2_ref, w21_ref, b21_ref, w22_ref, b22_ref)):
        hd = _conv_time(h, wd, bd, k=4, stride=2, pad=1)
        r = [jnp.maximum(a, 0.0) for a in hd]
        r = _conv_time(r, w1, b1, k=3, pad=1, relu_out=True)
        r = _conv_time(r, w2, b2, k=1)
        h = [a + b for a, b in zip(hd, r)]
    h = _conv_time(h, wo_ref, bo_ref, k=3, pad=1)
    for t in range(T_ENC):
        out_ref[t] = h[t]


def _run_encoder(x_tbc, enc_params, bblk):
    (w_in, b_in), blocks, (w_out, b_out) = enc_params
    wlist = []
    for w, b in ((w_in, b_in),
                 *[lw for blk in blocks for lw in blk],
                 (w_out, b_out)):
        wlist.append(jnp.transpose(w, (2, 1, 0)))
        wlist.append(b.reshape(1, -1))
    grid = (BATCH // bblk,)
    in_specs = [pl.BlockSpec((T_IN, bblk, IN_DIM), lambda i: (0, i, 0))]
    for w in wlist:
        in_specs.append(pl.BlockSpec(w.shape, lambda i, n=w.ndim: (0,) * n))
    out_spec = pl.BlockSpec((T_ENC, bblk, CODE_DIM), lambda i: (0, i, 0))
    return pl.pallas_call(
        _encoder_body,
        grid=grid,
        in_specs=in_specs,
        out_specs=out_spec,
        out_shape=jax.ShapeDtypeStruct((T_ENC, BATCH, CODE_DIM), jnp.float32),
    )(x_tbc, *wlist)


def _quant_body(xf_ref, cb_ref, idx_ref, xq_ref, *, tblk, cblk):
    xf = xf_ref[:]
    x2 = jnp.sum(xf * xf, axis=1, keepdims=True)
    xm2 = xf * (-2.0)
    ones = jnp.ones((tblk, 1), jnp.float32)
    xaug = jnp.concatenate([xm2, ones], axis=1)
    best = jnp.full((tblk, 1), jnp.inf, jnp.float32)
    barg = jnp.zeros((tblk, 1), jnp.int32)
    iota = jax.lax.broadcasted_iota(jnp.int32, (tblk, cblk), 1)
    for c in range(NB_CODE // cblk):
        cb = cb_ref[c * cblk:(c + 1) * cblk, :]
        c2 = jnp.sum(cb * cb, axis=1, keepdims=True)
        caug = jnp.concatenate([cb, c2], axis=1)
        dist = x2 + _mm_t(xaug, caug)
        lmin = jnp.min(dist, axis=1, keepdims=True)
        cand = jnp.where(dist == lmin, iota, cblk)
        larg = jnp.min(cand, axis=1, keepdims=True) + c * cblk
        better = lmin < best
        best = jnp.where(better, lmin, best)
        barg = jnp.where(better, larg, barg)
    idx_ref[:] = barg
    acc = None
    for c in range(NB_CODE // cblk):
        rel = barg - c * cblk
        oh = (iota == rel).astype(jnp.float32)
        p = _mm(oh, cb_ref[c * cblk:(c + 1) * cblk, :],
                prec=jax.lax.Precision.HIGHEST)
        acc = p if acc is None else acc + p
    xq_ref[:] = acc


def _run_quantizer(xf, codebook, tblk, cblk):
    ntok = xf.shape[0]
    grid = (ntok // tblk,)
    body = functools.partial(_quant_body, tblk=tblk, cblk=cblk)
    return pl.pallas_call(
        body,
        grid=grid,
        in_specs=[pl.BlockSpec((tblk, CODE_DIM), lambda i: (i, 0)),
                  pl.BlockSpec((NB_CODE, CODE_DIM), lambda i: (0, 0))],
        out_specs=[pl.BlockSpec((tblk, 1), lambda i: (i, 0)),
                   pl.BlockSpec((tblk, CODE_DIM), lambda i: (i, 0))],
        out_shape=[jax.ShapeDtypeStruct((ntok, 1), jnp.int32),
                   jax.ShapeDtypeStruct((ntok, CODE_DIM), jnp.float32)],
    )(xf, codebook)


def _decoder_body(x_ref, wi_ref, bi_ref,
                  w11_ref, b11_ref, w12_ref, b12_ref, wu1_ref, bu1_ref,
                  w21_ref, b21_ref, w22_ref, b22_ref, wu2_ref, bu2_ref,
                  wm_ref, bm_ref, wo_ref, bo_ref, out_ref):
    xs = [x_ref[t] for t in range(T_ENC)]
    h = _conv_time(xs, wi_ref, bi_ref, k=3, pad=1, relu_out=True)
    for (w1, b1, w2, b2, wu, bu) in (
            (w11_ref, b11_ref, w12_ref, b12_ref, wu1_ref, bu1_ref),
            (w21_ref, b21_ref, w22_ref, b22_ref, wu2_ref, bu2_ref)):
        r = [jnp.maximum(a, 0.0) for a in h]
        r = _conv_time(r, w1, b1, k=3, pad=1, relu_out=True)
        r = _conv_time(r, w2, b2, k=1)
        h = [a + b for a, b in zip(h, r)]
        h = [a for a in h for _ in range(2)]
        h = _conv_time(h, wu, bu, k=3, pad=1)
    h = _conv_time(h, wm_ref, bm_ref, k=3, pad=1, relu_out=True)
    h = _conv_time(h, wo_ref, bo_ref, k=3, pad=1)
    for t in range(T_IN):
        out_ref[t] = h[t]


def _run_decoder(xq_tbc, dec_params, bblk):
    (w_in, b_in), blocks, (w_mid, b_mid), (w_out, b_out) = dec_params
    wlist = []
    for w, b in ((w_in, b_in),
                 *[lw for blk in blocks for lw in blk],
                 (w_mid, b_mid), (w_out, b_out)):
        wlist.append(jnp.transpose(w, (2, 1, 0)))
        wlist.append(b.reshape(1, -1))
    grid = (BATCH // bblk,)
    in_specs = [pl.BlockSpec((T_ENC, bblk, CODE_DIM), lambda i: (0, i, 0))]
    for w in wlist:
        in_specs.append(pl.BlockSpec(w.shape, lambda i, n=w.ndim: (0,) * n))
    out_spec = pl.BlockSpec((T_IN, bblk, OUT_DIM), lambda i: (0, i, 0))
    return pl.pallas_call(
        _decoder_body,
        grid=grid,
        in_specs=in_specs,
        out_specs=out_spec,
        out_shape=jax.ShapeDtypeStruct((T_IN, BATCH, OUT_DIM), jnp.float32),
    )(xq_tbc, *wlist)


def _norm_rows(z):
    return z / (jnp.sqrt(jnp.sum(z * z, axis=1, keepdims=True)) + 1e-12)


def _sim_loss(zi_n, zj_n, mask):
    sim = _mm_t(zj_n, zi_n) / TEMP
    e = jnp.exp(sim)
    sp = jnp.sum(e * mask, axis=1, keepdims=True)
    sn = jnp.sum(e * (1.0 - mask), axis=1, keepdims=True)
    lp = jnp.log(sp / sn)
    rc = jnp.sum(mask, axis=1, keepdims=True)
    return -jnp.mean(rc * lp)


def _losses_body(te_ref, tw_ref, tb_ref, au_ref, g_ref, vc_ref, vr_ref,
                 o1_ref, o2_ref, o3_ref):
    b = te_ref.shape[0]
    text = _mm_t(te_ref[:], tw_ref[:]) + tb_ref[:]
    tn = _norm_rows(text)
    an = _norm_rows(au_ref[:])
    g1n = _norm_rows(g_ref[0])
    g2n = _norm_rows(g_ref[1])
    g3n = _norm_rows(g_ref[2])
    g4n = _norm_rows(g_ref[3])
    eye = (jax.lax.broadcasted_iota(jnp.int32, (b, b), 0)
           == jax.lax.broadcasted_iota(jnp.int32, (b, b), 1)).astype(jnp.float32)
    style = (vc_ref[:] == vr_ref[:]).astype(jnp.float32)
    o1_ref[:] = jnp.reshape(_sim_loss(tn, g1n, eye), (1, 1))
    o2_ref[:] = jnp.reshape(_sim_loss(an, g2n, eye), (1, 1))
    o3_ref[:] = jnp.reshape(_sim_loss(g3n, g4n, style), (1, 1))


def _run_losses(text_emb, text_W, text_b, audio, x_enc_tbc, vid):
    vid_i = vid.astype(jnp.int32)
    vc = vid_i.reshape(BATCH, 1)
    vr = vid_i.reshape(1, BATCH)
    scal = jax.ShapeDtypeStruct((1, 1), jnp.float32)
    return pl.pallas_call(
        _losses_body,
        out_shape=[scal, scal, scal],
    )(text_emb, text_W, text_b.reshape(1, -1), audio, x_enc_tbc, vc, vr)


def _stats_body(xf_ref, xq_ref, idx_ref, commit_ref, perp_ref, *, cblk):
    ntok = xf_ref.shape[0]
    d = xf_ref[:] - xq_ref[:]
    commit_ref[:] = jnp.reshape(jnp.sum(d * d) / (ntok * CODE_DIM), (1, 1))
    idx = idx_ref[:]
    ent = jnp.zeros((), jnp.float32)
    iota = jax.lax.broadcasted_iota(jnp.int32, (ntok, cblk), 1)
    inv = 1.0 / ntok
    for c in range(NB_CODE // cblk):
        eq = (idx == (iota + c * cblk)).astype(jnp.float32)
        cnt = jnp.sum(eq, axis=0)
        p = cnt * inv
        ent = ent + jnp.sum(p * jnp.log(p + 1e-10))
    perp_ref[:] = jnp.reshape(jnp.exp(-ent), (1, 1))


def _run_stats(xf, xq, idx, cblk=1024):
    scal = jax.ShapeDtypeStruct((1, 1), jnp.float32)
    body = functools.partial(_stats_body, cblk=cblk)
    return pl.pallas_call(body, out_shape=[scal, scal])(xf, xq, idx)


def kernel(gesture, audio_embeddings, text_embeddings, vid_indices, codebook,
           enc_params, dec_params, text_W, text_b):
    x_tbc = jnp.transpose(gesture.astype(jnp.float32), (1, 0, 2))
    x_enc = _run_encoder(x_tbc, enc_params, bblk=256)

    ntok = T_ENC * BATCH
    xf = x_enc.reshape(ntok, CODE_DIM)
    idx, xq = _run_quantizer(xf, codebook, tblk=512, cblk=1024)

    commit, perp = _run_stats(xf, xq, idx)

    gt_loss, ga_loss, gs_loss = _run_losses(
        text_embeddings, text_W, text_b, audio_embeddings, x_enc, vid_indices)

    xq_tbc = xq.reshape(T_ENC, BATCH, CODE_DIM)
    x_dec = _run_decoder(xq_tbc, dec_params, bblk=256)
    x_out = jnp.transpose(x_dec, (1, 0, 2))

    return (x_out, commit[0, 0], perp[0, 0],
            gt_loss[0, 0], ga_loss[0, 0], gs_loss[0, 0])


# speedup vs baseline: 4.5319x; 1.4867x over previous
import functools

import jax
import jax.numpy as jnp
from jax import lax
from jax.experimental import pallas as pl
from jax.experimental.pallas import tpu as pltpu
from jax.experimental.pallas import tpu_sc as plsc

CODE_DIM = 512
NB_CODE = 8192
WIDTH = 512
IN_DIM = 159
OUT_DIM = 159
TEMP = 0.1
BATCH = 1024
T_IN = 16
T_ENC = 4

_PREC = jax.lax.Precision.DEFAULT


def _mm(a, b, prec=_PREC):
    return jax.lax.dot_general(a, b, (((1,), (0,)), ((), ())),
                               preferred_element_type=jnp.float32,
                               precision=prec)


def _mm_t(a, b, prec=_PREC):
    return jax.lax.dot_general(a, b, (((1,), (1,)), ((), ())),
                               preferred_element_type=jnp.float32,
                               precision=prec)


def _conv_time(xs, w_ref, b_ref, k, stride=1, pad=0, relu_out=False):
    t_in = len(xs)
    t_out = (t_in + 2 * pad - k) // stride + 1
    outs = []
    bias = b_ref[:]
    for t in range(t_out):
        acc = None
        for j in range(k):
            ti = t * stride - pad + j
            if 0 <= ti < t_in:
                p = _mm(xs[ti], w_ref[j])
                acc = p if acc is None else acc + p
        acc = acc + bias
        if relu_out:
            acc = jnp.maximum(acc, 0.0)
        outs.append(acc)
    return outs


def _encoder_body(x_ref, wi_ref, bi_ref,
                  wd1_ref, bd1_ref, w11_ref, b11_ref, w12_ref, b12_ref,
                  wd2_ref, bd2_ref, w21_ref, b21_ref, w22_ref, b22_ref,
                  wo_ref, bo_ref, out_ref):
    xs = [x_ref[:, t, :] for t in range(T_IN)]
    h = _conv_time(xs, wi_ref, bi_ref, k=3, pad=1, relu_out=True)
    for (wd, bd, w1, b1, w2, b2) in (
            (wd1_ref, bd1_ref, w11_ref, b11_ref, w12_ref, b12_ref),
            (wd2_ref, bd2_ref, w21_ref, b21_ref, w22_ref, b22_ref)):
        hd = _conv_time(h, wd, bd, k=4, stride=2, pad=1)
        r = [jnp.maximum(a, 0.0) for a in hd]
        r = _conv_time(r, w1, b1, k=3, pad=1, relu_out=True)
        r = _conv_time(r, w2, b2, k=1)
        h = [a + b for a, b in zip(hd, r)]
    h = _conv_time(h, wo_ref, bo_ref, k=3, pad=1)
    for t in range(T_ENC):
        out_ref[t] = h[t]


def _run_encoder(gesture, enc_params, bblk):
    (w_in, b_in), blocks, (w_out, b_out) = enc_params
    wlist = []
    for w, b in ((w_in, b_in),
                 *[lw for blk in blocks for lw in blk],
                 (w_out, b_out)):
        wlist.append(jnp.transpose(w, (2, 1, 0)))
        wlist.append(b.reshape(1, -1))
    grid = (BATCH // bblk,)
    in_specs = [pl.BlockSpec((bblk, T_IN, IN_DIM), lambda i: (i, 0, 0))]
    for w in wlist:
        in_specs.append(pl.BlockSpec(w.shape, lambda i, n=w.ndim: (0,) * n))
    out_spec = pl.BlockSpec((T_ENC, bblk, CODE_DIM), lambda i: (0, i, 0))
    return pl.pallas_call(
        _encoder_body,
        grid=grid,
        in_specs=in_specs,
        out_specs=out_spec,
        out_shape=jax.ShapeDtypeStruct((T_ENC, BATCH, CODE_DIM), jnp.float32),
    )(gesture, *wlist)


def _quant_body(xf_ref, cb_ref, idx_ref, *, tblk, cblk):
    xf = xf_ref[:]
    ones_row = jnp.ones((1, CODE_DIM), jnp.float32)
    best = jnp.full((tblk, 1), jnp.inf, jnp.float32)
    barg = jnp.zeros((tblk, 1), jnp.int32)
    iota = jax.lax.broadcasted_iota(jnp.int32, (tblk, cblk), 1)
    for c in range(NB_CODE // cblk):
        cb = cb_ref[c * cblk:(c + 1) * cblk, :]
        c2 = _mm_t(ones_row, cb * cb)
        dist = c2 - 2.0 * _mm_t(xf, cb)
        lmin = jnp.min(dist, axis=1, keepdims=True)
        cand = jnp.where(dist == lmin, iota, cblk)
        larg = jnp.min(cand, axis=1, keepdims=True) + c * cblk
        better = lmin < best
        best = jnp.where(better, lmin, best)
        barg = jnp.where(better, larg, barg)
    idx_ref[:] = barg


def _run_quantizer(xf, codebook, tblk, cblk):
    ntok = xf.shape[0]
    grid = (ntok // tblk,)
    body = functools.partial(_quant_body, tblk=tblk, cblk=cblk)
    return pl.pallas_call(
        body,
        grid=grid,
        in_specs=[pl.BlockSpec((tblk, CODE_DIM), lambda i: (i, 0)),
                  pl.BlockSpec((NB_CODE, CODE_DIM), lambda i: (0, 0))],
        out_specs=pl.BlockSpec((tblk, 1), lambda i: (i, 0)),
        out_shape=jax.ShapeDtypeStruct((ntok, 1), jnp.int32),
    )(xf, codebook)


_SC_NC = 2
_SC_NS = 16


def _sc_gather(codebook, idx):
    ntok = idx.shape[0]
    nw = _SC_NC * _SC_NS
    bpw = ntok // nw
    mesh = plsc.VectorSubcoreMesh(core_axis_name="c", subcore_axis_name="s")

    @functools.partial(
        pl.kernel, mesh=mesh,
        out_type=jax.ShapeDtypeStruct((ntok, CODE_DIM), jnp.float32),
        scratch_types=[
            pltpu.VMEM((bpw,), jnp.int32),
            pltpu.VMEM((bpw, CODE_DIM), jnp.float32),
            pltpu.SemaphoreType.DMA,
        ],
    )
    def k(table_hbm, idx_hbm, out_hbm, idx_v, rows_v, sem):
        wid = lax.axis_index("s") * _SC_NC + lax.axis_index("c")
        base = wid * bpw
        pltpu.sync_copy(idx_hbm.at[pl.ds(base, bpw)], idx_v)
        pltpu.async_copy(table_hbm.at[idx_v], rows_v, sem).wait()
        pltpu.sync_copy(rows_v, out_hbm.at[pl.ds(base, bpw)])

    return k(codebook, idx)


def _decoder_body(x_ref, wi_ref, bi_ref,
                  w11_ref, b11_ref, w12_ref, b12_ref, wu1_ref, bu1_ref,
                  w21_ref, b21_ref, w22_ref, b22_ref, wu2_ref, bu2_ref,
                  wm_ref, bm_ref, wo_ref, bo_ref, out_ref):
    xs = [x_ref[t] for t in range(T_ENC)]
    h = _conv_time(xs, wi_ref, bi_ref, k=3, pad=1, relu_out=True)
    for (w1, b1, w2, b2, wu, bu) in (
            (w11_ref, b11_ref, w12_ref, b12_ref, wu1_ref, bu1_ref),
            (w21_ref, b21_ref, w22_ref, b22_ref, wu2_ref, bu2_ref)):
        r = [jnp.maximum(a, 0.0) for a in h]
        r = _conv_time(r, w1, b1, k=3, pad=1, relu_out=True)
        r = _conv_time(r, w2, b2, k=1)
        h = [a + b for a, b in zip(h, r)]
        h = [a for a in h for _ in range(2)]
        h = _conv_time(h, wu, bu, k=3, pad=1)
    h = _conv_time(h, wm_ref, bm_ref, k=3, pad=1, relu_out=True)
    h = _conv_time(h, wo_ref, bo_ref, k=3, pad=1)
    for t in range(T_IN):
        out_ref[:, t, :] = h[t]


def _run_decoder(xq_tbc, dec_params, bblk):
    (w_in, b_in), blocks, (w_mid, b_mid), (w_out, b_out) = dec_params
    wlist = []
    for w, b in ((w_in, b_in),
                 *[lw for blk in blocks for lw in blk],
                 (w_mid, b_mid), (w_out, b_out)):
        wlist.append(jnp.transpose(w, (2, 1, 0)))
        wlist.append(b.reshape(1, -1))
    grid = (BATCH // bblk,)
    in_specs = [pl.BlockSpec((T_ENC, bblk, CODE_DIM), lambda i: (0, i, 0))]
    for w in wlist:
        in_specs.append(pl.BlockSpec(w.shape, lambda i, n=w.ndim: (0,) * n))
    out_spec = pl.BlockSpec((bblk, T_IN, OUT_DIM), lambda i: (i, 0, 0))
    return pl.pallas_call(
        _decoder_body,
        grid=grid,
        in_specs=in_specs,
        out_specs=out_spec,
        out_shape=jax.ShapeDtypeStruct((BATCH, T_IN, OUT_DIM), jnp.float32),
    )(xq_tbc, *wlist)


def _norm_rows(z):
    return z / (jnp.sqrt(jnp.sum(z * z, axis=1, keepdims=True)) + 1e-12)


def _sim_loss(zi_n, zj_n, mask):
    sim = _mm_t(zj_n, zi_n) / TEMP
    e = jnp.exp(sim)
    sp = jnp.sum(e * mask, axis=1, keepdims=True)
    sn = jnp.sum(e * (1.0 - mask), axis=1, keepdims=True)
    lp = jnp.log(sp / sn)
    rc = jnp.sum(mask, axis=1, keepdims=True)
    return -jnp.mean(rc * lp)


def _losses_body(te_ref, tw_ref, tb_ref, au_ref, g_ref, vc_ref, vr_ref,
                 o1_ref, o2_ref, o3_ref):
    b = te_ref.shape[0]
    text = _mm_t(te_ref[:], tw_ref[:]) + tb_ref[:]
    tn = _norm_rows(text)
    an = _norm_rows(au_ref[:])
    g1n = _norm_rows(g_ref[0])
    g2n = _norm_rows(g_ref[1])
    g3n = _norm_rows(g_ref[2])
    g4n = _norm_rows(g_ref[3])
    eye = (jax.lax.broadcasted_iota(jnp.int32, (b, b), 0)
           == jax.lax.broadcasted_iota(jnp.int32, (b, b), 1)).astype(jnp.float32)
    style = (vc_ref[:] == vr_ref[:]).astype(jnp.float32)
    o1_ref[:] = jnp.reshape(_sim_loss(tn, g1n, eye), (1, 1))
    o2_ref[:] = jnp.reshape(_sim_loss(an, g2n, eye), (1, 1))
    o3_ref[:] = jnp.reshape(_sim_loss(g3n, g4n, style), (1, 1))


def _run_losses(text_emb, text_W, text_b, audio, x_enc_tbc, vid):
    vid_i = vid.astype(jnp.int32)
    vc = vid_i.reshape(BATCH, 1)
    vr = vid_i.reshape(1, BATCH)
    scal = jax.ShapeDtypeStruct((1, 1), jnp.float32)
    return pl.pallas_call(
        _losses_body,
        out_shape=[scal, scal, scal],
    )(text_emb, text_W, text_b.reshape(1, -1), audio, x_enc_tbc, vc, vr)


def _stats_body(xf_ref, xq_ref, idx_ref, commit_ref, perp_ref, *, cblk):
    ntok = xf_ref.shape[0]
    d = xf_ref[:] - xq_ref[:]
    commit_ref[:] = jnp.reshape(jnp.sum(d * d) / (ntok * CODE_DIM), (1, 1))
    idx = idx_ref[:]
    ent = jnp.zeros((), jnp.float32)
    iota = jax.lax.broadcasted_iota(jnp.int32, (ntok, cblk), 1)
    inv = 1.0 / ntok
    for c in range(NB_CODE // cblk):
        eq = (idx == (iota + c * cblk)).astype(jnp.float32)
        cnt = jnp.sum(eq, axis=0)
        p = cnt * inv
        ent = ent + jnp.sum(p * jnp.log(p + 1e-10))
    perp_ref[:] = jnp.reshape(jnp.exp(-ent), (1, 1))


def _run_stats(xf, xq, idx, cblk=1024):
    scal = jax.ShapeDtypeStruct((1, 1), jnp.float32)
    body = functools.partial(_stats_body, cblk=cblk)
    return pl.pallas_call(body, out_shape=[scal, scal])(xf, xq, idx)


def kernel(gesture, audio_embeddings, text_embeddings, vid_indices, codebook,
           enc_params, dec_params, text_W, text_b):
    x_enc = _run_encoder(gesture.astype(jnp.float32), enc_params, bblk=256)

    ntok = T_ENC * BATCH
    xf = x_enc.reshape(ntok, CODE_DIM)
    idx = _run_quantizer(xf, codebook, tblk=512, cblk=1024)

    xq = _sc_gather(codebook, idx.reshape(ntok))

    commit, perp = _run_stats(xf, xq, idx)

    gt_loss, ga_loss, gs_loss = _run_losses(
        text_embeddings, text_W, text_b, audio_embeddings, x_enc, vid_indices)

    xq_tbc = xq.reshape(T_ENC, BATCH, CODE_DIM)
    x_out = _run_decoder(xq_tbc, dec_params, bblk=256)

    return (x_out, commit[0, 0], perp[0, 0],
            gt_loss[0, 0], ga_loss[0, 0], gs_loss[0, 0])


# speedup vs baseline: 4.6382x; 1.0235x over previous
import functools

import jax
import jax.numpy as jnp
from jax import lax
from jax.experimental import pallas as pl
from jax.experimental.pallas import tpu as pltpu
from jax.experimental.pallas import tpu_sc as plsc

CODE_DIM = 512
NB_CODE = 8192
WIDTH = 512
IN_DIM = 159
OUT_DIM = 159
TEMP = 0.1
BATCH = 1024
T_IN = 16
T_ENC = 4

_PREC = jax.lax.Precision.DEFAULT


def _mm(a, b, prec=_PREC):
    return jax.lax.dot_general(a, b, (((1,), (0,)), ((), ())),
                               preferred_element_type=jnp.float32,
                               precision=prec)


def _mm_t(a, b, prec=_PREC):
    return jax.lax.dot_general(a, b, (((1,), (1,)), ((), ())),
                               preferred_element_type=jnp.float32,
                               precision=prec)


def _conv_time(xs, w_ref, b_ref, k, stride=1, pad=0, relu_out=False):
    t_in = len(xs)
    t_out = (t_in + 2 * pad - k) // stride + 1
    outs = []
    bias = b_ref[:]
    for t in range(t_out):
        acc = None
        for j in range(k):
            ti = t * stride - pad + j
            if 0 <= ti < t_in:
                p = _mm(xs[ti], w_ref[j])
                acc = p if acc is None else acc + p
        acc = acc + bias
        if relu_out:
            acc = jnp.maximum(acc, 0.0)
        outs.append(acc)
    return outs


def _encoder_body(x_ref, wi_ref, bi_ref,
                  wd1_ref, bd1_ref, w11_ref, b11_ref, w12_ref, b12_ref,
                  wd2_ref, bd2_ref, w21_ref, b21_ref, w22_ref, b22_ref,
                  wo_ref, bo_ref, out_ref):
    xs = [x_ref[:, t, :] for t in range(T_IN)]
    h = _conv_time(xs, wi_ref, bi_ref, k=3, pad=1, relu_out=True)
    for (wd, bd, w1, b1, w2, b2) in (
            (wd1_ref, bd1_ref, w11_ref, b11_ref, w12_ref, b12_ref),
            (wd2_ref, bd2_ref, w21_ref, b21_ref, w22_ref, b22_ref)):
        hd = _conv_time(h, wd, bd, k=4, stride=2, pad=1)
        r = [jnp.maximum(a, 0.0) for a in hd]
        r = _conv_time(r, w1, b1, k=3, pad=1, relu_out=True)
        r = _conv_time(r, w2, b2, k=1)
        h = [a + b for a, b in zip(hd, r)]
    h = _conv_time(h, wo_ref, bo_ref, k=3, pad=1)
    for t in range(T_ENC):
        out_ref[t] = h[t]


def _run_encoder(gesture, enc_params, bblk):
    (w_in, b_in), blocks, (w_out, b_out) = enc_params
    wlist = []
    for w, b in ((w_in, b_in),
                 *[lw for blk in blocks for lw in blk],
                 (w_out, b_out)):
        wlist.append(jnp.transpose(w, (2, 1, 0)))
        wlist.append(b.reshape(1, -1))
    grid = (BATCH // bblk,)
    in_specs = [pl.BlockSpec((bblk, T_IN, IN_DIM), lambda i: (i, 0, 0))]
    for w in wlist:
        in_specs.append(pl.BlockSpec(w.shape, lambda i, n=w.ndim: (0,) * n))
    out_spec = pl.BlockSpec((T_ENC, bblk, CODE_DIM), lambda i: (0, i, 0))
    return pl.pallas_call(
        _encoder_body,
        grid=grid,
        in_specs=in_specs,
        out_specs=out_spec,
        out_shape=jax.ShapeDtypeStruct((T_ENC, BATCH, CODE_DIM), jnp.float32),
    )(gesture, *wlist)


def _quant_body(xf_ref, cb_ref, idx_ref, *, tblk, cblk):
    xf = xf_ref[:]
    ones_row = jnp.ones((1, CODE_DIM), jnp.float32)
    best = jnp.full((tblk, 1), jnp.inf, jnp.float32)
    barg = jnp.zeros((tblk, 1), jnp.int32)
    iota = jax.lax.broadcasted_iota(jnp.int32, (tblk, cblk), 1)
    for c in range(NB_CODE // cblk):
        cb = cb_ref[c * cblk:(c + 1) * cblk, :]
        c2 = _mm_t(ones_row, cb * cb)
        dist = c2 - 2.0 * _mm_t(xf, cb)
        lmin = jnp.min(dist, axis=1, keepdims=True)
        cand = jnp.where(dist == lmin, iota, cblk)
        larg = jnp.min(cand, axis=1, keepdims=True) + c * cblk
        better = lmin < best
        best = jnp.where(better, lmin, best)
        barg = jnp.where(better, larg, barg)
    idx_ref[:] = barg


def _run_quantizer(xf, codebook, tblk, cblk):
    ntok = xf.shape[0]
    grid = (ntok // tblk,)
    body = functools.partial(_quant_body, tblk=tblk, cblk=cblk)
    return pl.pallas_call(
        body,
        grid=grid,
        in_specs=[pl.BlockSpec((tblk, CODE_DIM), lambda i: (i, 0)),
                  pl.BlockSpec((NB_CODE, CODE_DIM), lambda i: (0, 0))],
        out_specs=pl.BlockSpec((tblk, 1), lambda i: (i, 0)),
        out_shape=jax.ShapeDtypeStruct((ntok, 1), jnp.int32),
    )(xf, codebook)


_SC_NC = 2
_SC_NS = 16


def _sc_gather(codebook, idx):
    ntok = idx.shape[0]
    nw = _SC_NC * _SC_NS
    bpw = ntok // nw
    mesh = plsc.VectorSubcoreMesh(core_axis_name="c", subcore_axis_name="s")

    @functools.partial(
        pl.kernel, mesh=mesh,
        out_type=jax.ShapeDtypeStruct((ntok, CODE_DIM), jnp.float32),
        scratch_types=[
            pltpu.VMEM((bpw,), jnp.int32),
            pltpu.VMEM((bpw, CODE_DIM), jnp.float32),
            pltpu.SemaphoreType.DMA,
        ],
    )
    def k(table_hbm, idx_hbm, out_hbm, idx_v, rows_v, sem):
        wid = lax.axis_index("s") * _SC_NC + lax.axis_index("c")
        base = wid * bpw
        pltpu.sync_copy(idx_hbm.at[pl.ds(base, bpw)], idx_v)
        pltpu.async_copy(table_hbm.at[idx_v], rows_v, sem).wait()
        pltpu.sync_copy(rows_v, out_hbm.at[pl.ds(base, bpw)])

    return k(codebook, idx)


def _upconv_time(h, wu_ref, bu_ref):
    t_in = len(h)
    bias = bu_ref[:]
    w0 = wu_ref[0]
    w2 = wu_ref[2]
    w12 = wu_ref[1] + w2
    w01 = w0 + wu_ref[1]
    outs = []
    for i in range(t_in):
        e = _mm(h[i], w12)
        if i > 0:
            e = e + _mm(h[i - 1], w0)
        outs.append(e + bias)
        o = _mm(h[i], w01)
        if i + 1 < t_in:
            o = o + _mm(h[i + 1], w2)
        outs.append(o + bias)
    return outs


def _decoder_body(x_ref, wi_ref, bi_ref,
                  w11_ref, b11_ref, w12_ref, b12_ref, wu1_ref, bu1_ref,
                  w21_ref, b21_ref, w22_ref, b22_ref, wu2_ref, bu2_ref,
                  wm_ref, bm_ref, wo_ref, bo_ref, out_ref):
    xs = [x_ref[t] for t in range(T_ENC)]
    h = _conv_time(xs, wi_ref, bi_ref, k=3, pad=1, relu_out=True)
    for (w1, b1, w2, b2, wu, bu) in (
            (w11_ref, b11_ref, w12_ref, b12_ref, wu1_ref, bu1_ref),
            (w21_ref, b21_ref, w22_ref, b22_ref, wu2_ref, bu2_ref)):
        r = [jnp.maximum(a, 0.0) for a in h]
        r = _conv_time(r, w1, b1, k=3, pad=1, relu_out=True)
        r = _conv_time(r, w2, b2, k=1)
        h = [a + b for a, b in zip(h, r)]
        h = _upconv_time(h, wu, bu)
    h = _conv_time(h, wm_ref, bm_ref, k=3, pad=1, relu_out=True)
    h = _conv_time(h, wo_ref, bo_ref, k=3, pad=1)
    for t in range(T_IN):
        out_ref[:, t, :] = h[t]


def _run_decoder(xq_tbc, dec_params, bblk):
    (w_in, b_in), blocks, (w_mid, b_mid), (w_out, b_out) = dec_params
    wlist = []
    for w, b in ((w_in, b_in),
                 *[lw for blk in blocks for lw in blk],
                 (w_mid, b_mid), (w_out, b_out)):
        wlist.append(jnp.transpose(w, (2, 1, 0)))
        wlist.append(b.reshape(1, -1))
    grid = (BATCH // bblk,)
    in_specs = [pl.BlockSpec((T_ENC, bblk, CODE_DIM), lambda i: (0, i, 0))]
    for w in wlist:
        in_specs.append(pl.BlockSpec(w.shape, lambda i, n=w.ndim: (0,) * n))
    out_spec = pl.BlockSpec((bblk, T_IN, OUT_DIM), lambda i: (i, 0, 0))
    return pl.pallas_call(
        _decoder_body,
        grid=grid,
        in_specs=in_specs,
        out_specs=out_spec,
        out_shape=jax.ShapeDtypeStruct((BATCH, T_IN, OUT_DIM), jnp.float32),
    )(xq_tbc, *wlist)


def _norm_rows(z):
    return z / (jnp.sqrt(jnp.sum(z * z, axis=1, keepdims=True)) + 1e-12)


def _sim_loss(zi_n, zj_n, mask):
    sim = _mm_t(zj_n, zi_n) / TEMP
    e = jnp.exp(sim)
    sp = jnp.sum(e * mask, axis=1, keepdims=True)
    sn = jnp.sum(e * (1.0 - mask), axis=1, keepdims=True)
    lp = jnp.log(sp / sn)
    rc = jnp.sum(mask, axis=1, keepdims=True)
    return -jnp.mean(rc * lp)


def _losses_body(te_ref, tw_ref, tb_ref, au_ref, g_ref, vc_ref, vr_ref,
                 o1_ref, o2_ref, o3_ref):
    b = te_ref.shape[0]
    text = _mm_t(te_ref[:], tw_ref[:]) + tb_ref[:]
    tn = _norm_rows(text)
    an = _norm_rows(au_ref[:])
    g1n = _norm_rows(g_ref[0])
    g2n = _norm_rows(g_ref[1])
    g3n = _norm_rows(g_ref[2])
    g4n = _norm_rows(g_ref[3])
    eye = (jax.lax.broadcasted_iota(jnp.int32, (b, b), 0)
           == jax.lax.broadcasted_iota(jnp.int32, (b, b), 1)).astype(jnp.float32)
    style = (vc_ref[:] == vr_ref[:]).astype(jnp.float32)
    o1_ref[:] = jnp.reshape(_sim_loss(tn, g1n, eye), (1, 1))
    o2_ref[:] = jnp.reshape(_sim_loss(an, g2n, eye), (1, 1))
    o3_ref[:] = jnp.reshape(_sim_loss(g3n, g4n, style), (1, 1))


def _run_losses(text_emb, text_W, text_b, audio, x_enc_tbc, vid):
    vid_i = vid.astype(jnp.int32)
    vc = vid_i.reshape(BATCH, 1)
    vr = vid_i.reshape(1, BATCH)
    scal = jax.ShapeDtypeStruct((1, 1), jnp.float32)
    return pl.pallas_call(
        _losses_body,
        out_shape=[scal, scal, scal],
    )(text_emb, text_W, text_b.reshape(1, -1), audio, x_enc_tbc, vc, vr)


def _stats_body(xf_ref, xq_ref, idx_ref, commit_ref, perp_ref, *, cblk):
    ntok = xf_ref.shape[0]
    d = xf_ref[:] - xq_ref[:]
    commit_ref[:] = jnp.reshape(jnp.sum(d * d) / (ntok * CODE_DIM), (1, 1))
    idx = idx_ref[:]
    ent = jnp.zeros((), jnp.float32)
    iota = jax.lax.broadcasted_iota(jnp.int32, (ntok, cblk), 1)
    inv = 1.0 / ntok
    for c in range(NB_CODE // cblk):
        eq = (idx == (iota + c * cblk)).astype(jnp.float32)
        cnt = jnp.sum(eq, axis=0)
        p = cnt * inv
        ent = ent + jnp.sum(p * jnp.log(p + 1e-10))
    perp_ref[:] = jnp.reshape(jnp.exp(-ent), (1, 1))


def _run_stats(xf, xq, idx, cblk=1024):
    scal = jax.ShapeDtypeStruct((1, 1), jnp.float32)
    body = functools.partial(_stats_body, cblk=cblk)
    return pl.pallas_call(body, out_shape=[scal, scal])(xf, xq, idx)


def kernel(gesture, audio_embeddings, text_embeddings, vid_indices, codebook,
           enc_params, dec_params, text_W, text_b):
    x_enc = _run_encoder(gesture.astype(jnp.float32), enc_params, bblk=256)

    ntok = T_ENC * BATCH
    xf = x_enc.reshape(ntok, CODE_DIM)
    idx = _run_quantizer(xf, codebook, tblk=512, cblk=1024)

    xq = _sc_gather(codebook, idx.reshape(ntok))

    commit, perp = _run_stats(xf, xq, idx)

    gt_loss, ga_loss, gs_loss = _run_losses(
        text_embeddings, text_W, text_b, audio_embeddings, x_enc, vid_indices)

    xq_tbc = xq.reshape(T_ENC, BATCH, CODE_DIM)
    x_out = _run_decoder(xq_tbc, dec_params, bblk=256)

    return (x_out, commit[0, 0], perp[0, 0],
            gt_loss[0, 0], ga_loss[0, 0], gs_loss[0, 0])
